# baseline (speedup 1.0000x reference)
import functools
import math

import jax
import jax.numpy as jnp
from jax import lax
from jax.experimental import pallas as pl
from jax.experimental.pallas import tpu as pltpu
from jax.experimental.pallas import tpu_sc as plsc

D = 128
L_SEQ = 200
BATCH = 1024
RPC = 128
N_CHUNKS = 2 * L_SEQ * (BATCH // RPC)
NW = 32
CHUNKS_PER_W = N_CHUNKS // NW
CHUNKS_PER_L = BATCH // RPC
LANES = 16


L_PAD = 224


def _pe_body(out_ref):
    pos = lax.broadcasted_iota(jnp.int32, (L_PAD, D), 0).astype(jnp.float32)
    d = lax.broadcasted_iota(jnp.int32, (L_PAD, D), 1)
    d_even = (d // 2) * 2
    div = jnp.exp(d_even.astype(jnp.float32) * (-math.log(10000.0) / D))
    angle = pos * div
    out_ref[...] = jnp.where(d % 2 == 0, jnp.sin(angle), jnp.cos(angle))


def _compute_pe():
    return pl.pallas_call(
        _pe_body,
        out_shape=jax.ShapeDtypeStruct((L_PAD, D), jnp.float32),
    )()


NB = 5
DP = NB - 2


def _sc_body(idx_hbm, pe_hbm, table0, table1, out_hbm, idx_v, rows_v, pe_v,
             gsem, osem):
    wid = lax.axis_index("s") * 2 + lax.axis_index("c")
    m0 = lax.rem(wid, 16) * CHUNKS_PER_W
    N = CHUNKS_PER_W
    l_raw = m0 // CHUNKS_PER_L
    l_base = pl.multiple_of(l_raw - lax.rem(l_raw, 8), 8)
    pltpu.sync_copy(pe_hbm.at[pl.ds(l_base, 24)], pe_v)
    off = lax.rem(m0, 8)

    def run(table, t_base):
        astart = pl.multiple_of(t_base + m0 - off, 8)
        pltpu.sync_copy(idx_hbm.at[pl.ds(astart, CHUNKS_PER_W + 4)], idx_v)

        def gather(k, b):
            pltpu.async_copy(table.at[idx_v.at[k + off]], rows_v.at[b],
                             gsem.at[b])

        def wait_out(b):
            pltpu.make_async_copy(rows_v.at[b], out_hbm.at[0], osem.at[b]).wait()

        def process(k, b):
            pltpu.make_async_copy(
                table.at[idx_v.at[k + off]], rows_v.at[b], gsem.at[b]).wait()
            m = m0 + k
            ell = m // CHUNKS_PER_L - l_base
            pe_row = [pe_v[ell, pl.ds(j * LANES, LANES)]
                      for j in range(D // LANES)]

            @plsc.parallel_loop(0, RPC, step=1, unroll=4)
            def _(r):
                for j in range(D // LANES):
                    sl = pl.ds(j * LANES, LANES)
                    rows_v[b, r, sl] = rows_v[b, r, sl] + pe_row[j]

            pltpu.async_copy(rows_v.at[b], out_hbm.at[t_base + m], osem.at[b])

        for k in range(DP):
            gather(k, k)
        for k in range(NB):
            pb = (k + DP) % NB
            if k + DP >= NB:
                wait_out(pb)
            gather(k + DP, pb)
            process(k, k)

        def group(g, _):
            for b in range(NB):
                k = g * NB + b
                pb = (b + DP) % NB
                wait_out(pb)
                gather(k + DP, pb)
                process(k, b)
            return ()

        lax.fori_loop(1, N // NB - 1, group, ())

        for b in range(NB):
            k = N - NB + b
            if k + DP < N:
                pb = (b + DP) % NB
                wait_out(pb)
                gather(k + DP, pb)
            process(k, b)
        for b in range(NB):
            wait_out(b)

    @pl.when(wid < 16)
    def _():
        run(table0, 0)

    @pl.when(wid >= 16)
    def _():
        run(table1, N_CHUNKS // 2)


def _sc_gather(idx, pe, src_table, tgt_table):
    mesh = plsc.VectorSubcoreMesh(core_axis_name="c", subcore_axis_name="s")
    return pl.kernel(
        _sc_body,
        out_type=jax.ShapeDtypeStruct((N_CHUNKS, RPC, D), jnp.float32),
        mesh=mesh,
        scratch_types=[
            pltpu.VMEM((CHUNKS_PER_W + 4, RPC), jnp.int32),
            pltpu.VMEM((NB, RPC, D), jnp.float32),
            pltpu.VMEM((24, D), jnp.float32),
            pltpu.SemaphoreType.DMA((NB,)),
            pltpu.SemaphoreType.DMA((NB,)),
        ],
    )(idx, pe, src_table, tgt_table)


def kernel(src, tgt, src_table, tgt_table):
    idx = jnp.concatenate(
        [src.reshape(-1), tgt.reshape(-1)]
    ).astype(jnp.int32).reshape(N_CHUNKS, RPC)
    pe = _compute_pe()
    out = _sc_gather(idx, pe, src_table, tgt_table)
    return out.reshape(2, L_SEQ, BATCH, D)


# speedup vs baseline: 1.0045x; 1.0045x over previous
import functools
import math

import jax
import jax.numpy as jnp
from jax import lax
from jax.experimental import pallas as pl
from jax.experimental.pallas import tpu as pltpu
from jax.experimental.pallas import tpu_sc as plsc

D = 128
L_SEQ = 200
BATCH = 1024
RPC = 128
N_CHUNKS = 2 * L_SEQ * (BATCH // RPC)
NW = 32
CHUNKS_PER_W = N_CHUNKS // NW
CHUNKS_PER_L = BATCH // RPC
LANES = 16


L_PAD = 224


def _pe_body(out_ref):
    pos = lax.broadcasted_iota(jnp.int32, (L_PAD, D), 0).astype(jnp.float32)
    d = lax.broadcasted_iota(jnp.int32, (L_PAD, D), 1)
    d_even = (d // 2) * 2
    div = jnp.exp(d_even.astype(jnp.float32) * (-math.log(10000.0) / D))
    angle = pos * div
    out_ref[...] = jnp.where(d % 2 == 0, jnp.sin(angle), jnp.cos(angle))


def _compute_pe():
    return pl.pallas_call(
        _pe_body,
        out_shape=jax.ShapeDtypeStruct((L_PAD, D), jnp.float32),
    )()


NB = 5
DP = 2


def _sc_body(idx_hbm, pe_hbm, table0, table1, out_hbm, idx_v, rows_v, pe_v,
             gsem, osem):
    wid = lax.axis_index("s") * 2 + lax.axis_index("c")
    m0 = lax.rem(wid, 16) * CHUNKS_PER_W
    N = CHUNKS_PER_W
    l_raw = m0 // CHUNKS_PER_L
    l_base = pl.multiple_of(l_raw - lax.rem(l_raw, 8), 8)
    pltpu.sync_copy(pe_hbm.at[pl.ds(l_base, 24)], pe_v)
    off = lax.rem(m0, 8)

    def run(table, t_base):
        astart = pl.multiple_of(t_base + m0 - off, 8)
        pltpu.sync_copy(idx_hbm.at[pl.ds(astart, CHUNKS_PER_W + 4)], idx_v)

        def gather(k, b):
            pltpu.async_copy(table.at[idx_v.at[k + off]], rows_v.at[b],
                             gsem.at[b])

        def wait_out(b):
            pltpu.make_async_copy(rows_v.at[b], out_hbm.at[0], osem.at[b]).wait()

        def process(k, b):
            pltpu.make_async_copy(
                table.at[idx_v.at[k + off]], rows_v.at[b], gsem.at[b]).wait()
            m = m0 + k
            ell = m // CHUNKS_PER_L - l_base
            pe_row = [pe_v[ell, pl.ds(j * LANES, LANES)]
                      for j in range(D // LANES)]

            @plsc.parallel_loop(0, RPC, step=1, unroll=4)
            def _(r):
                for j in range(D // LANES):
                    sl = pl.ds(j * LANES, LANES)
                    rows_v[b, r, sl] = rows_v[b, r, sl] + pe_row[j]

            pltpu.async_copy(rows_v.at[b], out_hbm.at[t_base + m], osem.at[b])

        for k in range(DP):
            gather(k, k)
        for k in range(NB):
            pb = (k + DP) % NB
            if k + DP >= NB:
                wait_out(pb)
            gather(k + DP, pb)
            process(k, k)

        def group(g, _):
            for b in range(NB):
                k = g * NB + b
                pb = (b + DP) % NB
                wait_out(pb)
                gather(k + DP, pb)
                process(k, b)
            return ()

        lax.fori_loop(1, N // NB - 1, group, ())

        for b in range(NB):
            k = N - NB + b
            if k + DP < N:
                pb = (b + DP) % NB
                wait_out(pb)
                gather(k + DP, pb)
            process(k, b)
        for b in range(NB):
            wait_out(b)

    @pl.when(wid < 16)
    def _():
        run(table0, 0)

    @pl.when(wid >= 16)
    def _():
        run(table1, N_CHUNKS // 2)


def _sc_gather(idx, pe, src_table, tgt_table):
    mesh = plsc.VectorSubcoreMesh(core_axis_name="c", subcore_axis_name="s")
    return pl.kernel(
        _sc_body,
        out_type=jax.ShapeDtypeStruct((N_CHUNKS, RPC, D), jnp.float32),
        mesh=mesh,
        scratch_types=[
            pltpu.VMEM((CHUNKS_PER_W + 4, RPC), jnp.int32),
            pltpu.VMEM((NB, RPC, D), jnp.float32),
            pltpu.VMEM((24, D), jnp.float32),
            pltpu.SemaphoreType.DMA((NB,)),
            pltpu.SemaphoreType.DMA((NB,)),
        ],
    )(idx, pe, src_table, tgt_table)


def kernel(src, tgt, src_table, tgt_table):
    idx = jnp.concatenate(
        [src.T.reshape(-1), tgt.T.reshape(-1)]
    ).astype(jnp.int32).reshape(N_CHUNKS, RPC)
    pe = _compute_pe()
    out = _sc_gather(idx, pe, src_table, tgt_table)
    return out.reshape(2, L_SEQ, BATCH, D)


# speedup vs baseline: 1.0297x; 1.0250x over previous
import functools
import math

import jax
import jax.numpy as jnp
from jax import lax
from jax.experimental import pallas as pl
from jax.experimental.pallas import tpu as pltpu
from jax.experimental.pallas import tpu_sc as plsc

D = 128
L_SEQ = 200
BATCH = 1024
RPC = 128
N_CHUNKS = 2 * L_SEQ * (BATCH // RPC)
NW = 32
CHUNKS_PER_W = N_CHUNKS // NW
CHUNKS_PER_L = BATCH // RPC
LANES = 16


L_PAD = 224


def _pe_body(out_ref):
    pos = lax.broadcasted_iota(jnp.int32, (L_PAD, D), 0).astype(jnp.float32)
    d = lax.broadcasted_iota(jnp.int32, (L_PAD, D), 1)
    d_even = (d // 2) * 2
    div = jnp.exp(d_even.astype(jnp.float32) * (-math.log(10000.0) / D))
    angle = pos * div
    out_ref[...] = jnp.where(d % 2 == 0, jnp.sin(angle), jnp.cos(angle))


def _compute_pe():
    return pl.pallas_call(
        _pe_body,
        out_shape=jax.ShapeDtypeStruct((L_PAD, D), jnp.float32),
    )()


NB = 5
DP = NB - 2


def _sc_body(idx_hbm, pe_hbm, table0, table1, out_hbm, idx_v, rows_v, pe_v,
             sp_v, gsem, osem):
    sid = lax.axis_index("s")
    wid = lax.axis_index("s") * 2 + lax.axis_index("c")
    m0 = lax.rem(wid, 16) * CHUNKS_PER_W
    N = CHUNKS_PER_W
    l_raw = m0 // CHUNKS_PER_L
    l_base = pl.multiple_of(l_raw - lax.rem(l_raw, 8), 8)
    pltpu.sync_copy(pe_hbm.at[pl.ds(l_base, 24)], pe_v)
    off = lax.rem(m0, 8)

    def run(table, t_base):
        astart = pl.multiple_of(t_base + m0 - off, 8)
        pltpu.sync_copy(idx_hbm.at[pl.ds(astart, CHUNKS_PER_W + 4)], idx_v)

        def gather(k, b):
            pltpu.async_copy(table.at[idx_v.at[k + off]], rows_v.at[b],
                             gsem.at[b])

        def process(k, b, first):
            pltpu.make_async_copy(
                table.at[idx_v.at[k + off]], rows_v.at[b], gsem.at[b]).wait()
            m = m0 + k
            ell = m // CHUNKS_PER_L - l_base
            pe_row = [pe_v[ell, pl.ds(j * LANES, LANES)]
                      for j in range(D // LANES)]

            @plsc.parallel_loop(0, RPC, step=1, unroll=4)
            def _(r):
                for j in range(D // LANES):
                    sl = pl.ds(j * LANES, LANES)
                    rows_v[b, r, sl] = rows_v[b, r, sl] + pe_row[j]

            if not first:
                pltpu.make_async_copy(
                    sp_v.at[sid, b % 2], out_hbm.at[0], osem.at[b % 2]).wait()
            pltpu.sync_copy(rows_v.at[b], sp_v.at[sid, b % 2])
            pltpu.async_copy(sp_v.at[sid, b % 2], out_hbm.at[t_base + m],
                             osem.at[b % 2])

        for k in range(DP):
            gather(k, k)
        for k in range(NB):
            gather(k + DP, (k + DP) % NB)
            process(k, k, k < 2)

        def group(g, _):
            for b in range(NB):
                k = g * NB + b
                gather(k + DP, (b + DP) % NB)
                process(k, b, False)
            return ()

        lax.fori_loop(1, N // NB - 1, group, ())

        for b in range(NB):
            k = N - NB + b
            if k + DP < N:
                gather(k + DP, (b + DP) % NB)
            process(k, b, False)
        for b in range(2):
            pltpu.make_async_copy(sp_v.at[sid, b], out_hbm.at[0],
                                  osem.at[b]).wait()

    @pl.when(wid < 16)
    def _():
        run(table0, 0)

    @pl.when(wid >= 16)
    def _():
        run(table1, N_CHUNKS // 2)


def _sc_gather(idx, pe, src_table, tgt_table):
    mesh = plsc.VectorSubcoreMesh(core_axis_name="c", subcore_axis_name="s")
    return pl.kernel(
        _sc_body,
        out_type=jax.ShapeDtypeStruct((N_CHUNKS, RPC, D), jnp.float32),
        mesh=mesh,
        scratch_types=[
            pltpu.VMEM((CHUNKS_PER_W + 4, RPC), jnp.int32),
            pltpu.VMEM((NB, RPC, D), jnp.float32),
            pltpu.VMEM((24, D), jnp.float32),
            pltpu.VMEM_SHARED((16, 2, RPC, D), jnp.float32),
            pltpu.SemaphoreType.DMA((NB,)),
            pltpu.SemaphoreType.DMA((2,)),
        ],
    )(idx, pe, src_table, tgt_table)


def kernel(src, tgt, src_table, tgt_table):
    idx = jnp.concatenate(
        [src.T.reshape(-1), tgt.T.reshape(-1)]
    ).astype(jnp.int32).reshape(N_CHUNKS, RPC)
    pe = _compute_pe()
    out = _sc_gather(idx, pe, src_table, tgt_table)
    return out.reshape(2, L_SEQ, BATCH, D)
